# Initial kernel scaffold; baseline (speedup 1.0000x reference)
import jax
import jax.numpy as jnp
from jax.experimental import pallas as pl


def kernel(x, edge_index, W1_l, b1_l, W1_r, W2_l, b2_l, W2_r):
    raise NotImplementedError("write your pallas kernel here")


# speedup vs baseline: 5.9567x; 5.9567x over previous
import functools

import jax
import jax.numpy as jnp
from jax import lax
from jax.experimental import pallas as pl
from jax.experimental.pallas import tpu as pltpu
from jax.experimental.pallas import tpu_sc as plsc

N_NODES = 10000
N_EDGES = 320000
D = 128
L = 16

NC = 2
NS = 16
NW = NC * NS
EPW = N_EDGES // NW
CHUNK = 80
NCHUNK = EPW // CHUNK
INIT_TILES = 10
ROWS_PER_TILE = N_NODES // INIT_TILES
STAGE = 40
DSTAGE = 200


def _sc_deg(dst2, zdeg, ones2):
    mesh = plsc.VectorSubcoreMesh(core_axis_name="c", subcore_axis_name="s")

    @functools.partial(
        pl.kernel,
        out_type=jax.ShapeDtypeStruct((NC, N_NODES, D), jnp.float32),
        mesh=mesh,
        scratch_types=[
            pltpu.VMEM((NCHUNK, CHUNK), jnp.int32),
            pltpu.VMEM((CHUNK, D), jnp.float32),
            pltpu.VMEM((STAGE, D), jnp.float32),
            pltpu.VMEM_SHARED((N_NODES, D), jnp.float32),
        ],
    )
    def k(dst_hbm, zdeg_hbm, ones_hbm, deg_out, didx, ones_v, dstage,
          deg_sh):
        cid = lax.axis_index("c")
        sid = lax.axis_index("s")
        wid = cid * NS + sid

        @pl.when(sid < INIT_TILES)
        def _():
            pltpu.sync_copy(zdeg_hbm, dstage)
            for c in range(ROWS_PER_TILE // STAGE):
                pltpu.sync_copy(
                    dstage,
                    deg_sh.at[pl.ds(sid * ROWS_PER_TILE + c * STAGE,
                                    STAGE)])

        pltpu.sync_copy(ones_hbm, ones_v)
        pltpu.sync_copy(dst_hbm.at[wid], didx)
        plsc.subcore_barrier()

        def step(j, carry):
            pltpu.sync_copy(ones_v, deg_sh.at[didx.at[j]], add=True)
            return carry

        lax.fori_loop(0, NCHUNK, step, 0)
        plsc.subcore_barrier()

        @pl.when(sid < INIT_TILES)
        def _():
            for c in range(ROWS_PER_TILE // STAGE):
                base = sid * ROWS_PER_TILE + c * STAGE
                pltpu.sync_copy(deg_sh.at[pl.ds(base, STAGE)], dstage)
                pltpu.sync_copy(dstage, deg_out.at[cid, pl.ds(base, STAGE)])

    return k(dst2, zdeg, ones2)


def _sc_agg(x, src2, dst2, zrow):
    mesh = plsc.VectorSubcoreMesh(core_axis_name="c", subcore_axis_name="s")

    @functools.partial(
        pl.kernel,
        out_type=jax.ShapeDtypeStruct((NC, N_NODES, D), jnp.float32),
        mesh=mesh,
        scratch_types=[
            pltpu.VMEM((NCHUNK, CHUNK), jnp.int32),
            pltpu.VMEM((NCHUNK, CHUNK), jnp.int32),
            pltpu.VMEM((CHUNK, D), jnp.float32),
            pltpu.VMEM((STAGE, D), jnp.float32),
            pltpu.VMEM_SHARED((N_NODES, D), jnp.float32),
            pltpu.SemaphoreType.DMA,
        ],
    )
    def k(x_hbm, src_hbm, dst_hbm, zrow_hbm, agg_out, sidx, didx, rows,
          stage, agg_sh, gsem):
        cid = lax.axis_index("c")
        sid = lax.axis_index("s")
        wid = cid * NS + sid

        @pl.when(sid < INIT_TILES)
        def _():
            pltpu.sync_copy(zrow_hbm, stage)
            for c in range(ROWS_PER_TILE // STAGE):
                pltpu.sync_copy(
                    stage,
                    agg_sh.at[pl.ds(sid * ROWS_PER_TILE + c * STAGE, STAGE)])

        pltpu.sync_copy(src_hbm.at[wid], sidx)
        pltpu.sync_copy(dst_hbm.at[wid], didx)
        plsc.subcore_barrier()

        def step(j, carry):
            pltpu.async_copy(x_hbm.at[sidx.at[j]], rows, gsem).wait()
            pltpu.sync_copy(rows, agg_sh.at[didx.at[j]], add=True)
            return carry

        lax.fori_loop(0, NCHUNK, step, 0)
        plsc.subcore_barrier()

        @pl.when(sid < INIT_TILES)
        def _():
            for c in range(ROWS_PER_TILE // STAGE):
                base = sid * ROWS_PER_TILE + c * STAGE
                pltpu.sync_copy(agg_sh.at[pl.ds(base, STAGE)], stage)
                pltpu.sync_copy(stage, agg_out.at[cid, pl.ds(base, STAGE)])

    return k(x, src2, dst2, zrow)


def _tc_layer(aggp, degp, x_in, W_l, b_l, W_r, relu):
    BLK = 1000
    grid = (N_NODES // BLK,)

    def body(a0, a1, d0, d1, xr, wl, bl, wr, o):
        deg = jnp.maximum(d0[...] + d1[...], 1.0)
        agg = (a0[...] + a1[...]) / deg
        acc = lax.dot_general(agg, wl[...], (((1,), (1,)), ((), ())),
                              preferred_element_type=jnp.float32)
        acc = acc + lax.dot_general(xr[...], wr[...], (((1,), (1,)), ((), ())),
                                    preferred_element_type=jnp.float32)
        acc = acc + bl[...]
        if relu:
            acc = jnp.maximum(acc, 0.0)
        o[...] = acc

    d0 = degp[0, :, 0:1]
    d1 = degp[1, :, 0:1]
    return pl.pallas_call(
        body,
        grid=grid,
        in_specs=[
            pl.BlockSpec((BLK, D), lambda i: (i, 0)),
            pl.BlockSpec((BLK, D), lambda i: (i, 0)),
            pl.BlockSpec((BLK, 1), lambda i: (i, 0)),
            pl.BlockSpec((BLK, 1), lambda i: (i, 0)),
            pl.BlockSpec((BLK, D), lambda i: (i, 0)),
            pl.BlockSpec((D, D), lambda i: (0, 0)),
            pl.BlockSpec((1, D), lambda i: (0, 0)),
            pl.BlockSpec((D, D), lambda i: (0, 0)),
        ],
        out_specs=pl.BlockSpec((BLK, D), lambda i: (i, 0)),
        out_shape=jax.ShapeDtypeStruct((N_NODES, D), jnp.float32),
    )(aggp[0], aggp[1], d0, d1, x_in, W_l, b_l.reshape(1, D), W_r)


def kernel(x, edge_index, W1_l, b1_l, W1_r, W2_l, b2_l, W2_r):
    ei = edge_index.astype(jnp.int32)
    src2 = ei[0].reshape(NW, NCHUNK, CHUNK)
    dst2 = ei[1].reshape(NW, NCHUNK, CHUNK)
    zrow = jnp.zeros((STAGE, D), jnp.float32)
    zdeg = jnp.zeros((STAGE, D), jnp.float32)
    ones2 = jnp.ones((CHUNK, D), jnp.float32)

    degp = _sc_deg(dst2, zdeg, ones2)
    aggp1 = _sc_agg(x, src2, dst2, zrow)
    h = _tc_layer(aggp1, degp, x, W1_l, b1_l, W1_r, relu=True)
    aggp2 = _sc_agg(h, src2, dst2, zrow)
    out = _tc_layer(aggp2, degp, h, W2_l, b2_l, W2_r, relu=False)
    return out


# speedup vs baseline: 8.5194x; 1.4302x over previous
import functools

import jax
import jax.numpy as jnp
from jax import lax
from jax.experimental import pallas as pl
from jax.experimental.pallas import tpu as pltpu
from jax.experimental.pallas import tpu_sc as plsc

N_NODES = 10000
N_EDGES = 320000
D = 128
L = 16

NC = 2
NS = 16
NW = NC * NS
EPW = N_EDGES // NW
CHUNK = 80
NCHUNK = EPW // CHUNK
INIT_TILES = 10
ROWS_PER_TILE = N_NODES // INIT_TILES
STAGE = 40


def _sc_deg(dst2, zdeg, ones2):
    mesh = plsc.VectorSubcoreMesh(core_axis_name="c", subcore_axis_name="s")

    @functools.partial(
        pl.kernel,
        out_type=jax.ShapeDtypeStruct((NC, N_NODES, D), jnp.float32),
        mesh=mesh,
        scratch_types=[
            pltpu.VMEM((NCHUNK, CHUNK), jnp.int32),
            pltpu.VMEM((CHUNK, D), jnp.float32),
            pltpu.VMEM((STAGE, D), jnp.float32),
            pltpu.VMEM_SHARED((N_NODES, D), jnp.float32),
        ],
    )
    def k(dst_hbm, zdeg_hbm, ones_hbm, deg_out, didx, ones_v, dstage,
          deg_sh):
        cid = lax.axis_index("c")
        sid = lax.axis_index("s")
        wid = cid * NS + sid

        @pl.when(sid < INIT_TILES)
        def _():
            pltpu.sync_copy(zdeg_hbm, dstage)
            for c in range(ROWS_PER_TILE // STAGE):
                pltpu.sync_copy(
                    dstage,
                    deg_sh.at[pl.ds(sid * ROWS_PER_TILE + c * STAGE,
                                    STAGE)])

        pltpu.sync_copy(ones_hbm, ones_v)
        pltpu.sync_copy(dst_hbm.at[wid], didx)
        plsc.subcore_barrier()

        def step(j, carry):
            pltpu.sync_copy(ones_v, deg_sh.at[didx.at[j]], add=True)
            return carry

        lax.fori_loop(0, NCHUNK, step, 0)
        plsc.subcore_barrier()

        @pl.when(sid < INIT_TILES)
        def _():
            for c in range(ROWS_PER_TILE // STAGE):
                base = sid * ROWS_PER_TILE + c * STAGE
                pltpu.sync_copy(deg_sh.at[pl.ds(base, STAGE)], dstage)
                pltpu.sync_copy(dstage, deg_out.at[cid, pl.ds(base, STAGE)])

    return k(dst2, zdeg, ones2)


def _sc_agg(x, packed2, zrow):
    mesh = plsc.VectorSubcoreMesh(core_axis_name="c", subcore_axis_name="s")

    @functools.partial(
        pl.kernel,
        out_type=jax.ShapeDtypeStruct((NC, N_NODES, D), jnp.float32),
        mesh=mesh,
        scratch_types=[
            pltpu.VMEM((NCHUNK, CHUNK), jnp.int32),
            pltpu.VMEM((2, CHUNK), jnp.int32),
            pltpu.VMEM((2, CHUNK), jnp.int32),
            pltpu.VMEM((CHUNK, D), jnp.float32),
            pltpu.VMEM((CHUNK, D), jnp.float32),
            pltpu.VMEM_SHARED((N_NODES, D), jnp.float32),
            pltpu.SemaphoreType.DMA,
            pltpu.SemaphoreType.DMA,
        ],
    )
    def k(x_hbm, pk_hbm, zrow_hbm, agg_out, pidx, usrc, udst, rows0,
          rows1, agg_sh, gsem0, gsem1):
        cid = lax.axis_index("c")
        sid = lax.axis_index("s")
        wid = cid * NS + sid

        def unpack(j, b):
            for kk in range(CHUNK // L):
                v = pidx[j, pl.ds(kk * L, L)]
                usrc[b, pl.ds(kk * L, L)] = lax.bitwise_and(v, 0xFFFF)
                udst[b, pl.ds(kk * L, L)] = lax.shift_right_logical(v, 16)

        @pl.when(sid < INIT_TILES)
        def _():
            pltpu.sync_copy(zrow_hbm, rows0.at[pl.ds(0, STAGE)])
            for c in range(ROWS_PER_TILE // STAGE):
                pltpu.sync_copy(
                    rows0.at[pl.ds(0, STAGE)],
                    agg_sh.at[pl.ds(sid * ROWS_PER_TILE + c * STAGE, STAGE)])

        pltpu.sync_copy(pk_hbm.at[wid], pidx)
        unpack(0, 0)
        unpack(1, 1)
        pltpu.async_copy(x_hbm.at[usrc.at[0]], rows0, gsem0)
        pltpu.async_copy(x_hbm.at[usrc.at[1]], rows1, gsem1)
        plsc.subcore_barrier()

        def step(i, carry):
            for b, (rb, sb) in enumerate(((rows0, gsem0), (rows1, gsem1))):
                j = 2 * i + b
                pltpu.make_async_copy(x_hbm.at[usrc.at[b]], rb, sb).wait()
                pltpu.sync_copy(rb, agg_sh.at[udst.at[b]], add=True)
                jn = jnp.minimum(j + 2, NCHUNK - 1)
                unpack(jn, b)
                pltpu.async_copy(x_hbm.at[usrc.at[b]], rb, sb)
            return carry

        lax.fori_loop(0, NCHUNK // 2, step, 0)
        pltpu.make_async_copy(x_hbm.at[usrc.at[0]], rows0, gsem0).wait()
        pltpu.sync_copy(rows0, agg_sh.at[udst.at[0]], add=True)
        pltpu.make_async_copy(x_hbm.at[usrc.at[1]], rows1, gsem1).wait()
        plsc.subcore_barrier()

        @pl.when(sid < INIT_TILES)
        def _():
            for c in range(ROWS_PER_TILE // STAGE):
                base = sid * ROWS_PER_TILE + c * STAGE
                pltpu.sync_copy(agg_sh.at[pl.ds(base, STAGE)],
                                rows0.at[pl.ds(0, STAGE)])
                pltpu.sync_copy(rows0.at[pl.ds(0, STAGE)],
                                agg_out.at[cid, pl.ds(base, STAGE)])

    return k(x, packed2, zrow)


def _tc_layer(aggp, degp, x_in, W_l, b_l, W_r, relu):
    BLK = 1000
    grid = (N_NODES // BLK,)

    def body(a0, a1, d0, d1, xr, wl, bl, wr, o):
        deg = jnp.maximum(d0[...] + d1[...], 1.0)
        agg = (a0[...] + a1[...]) / deg
        acc = lax.dot_general(agg, wl[...], (((1,), (1,)), ((), ())),
                              preferred_element_type=jnp.float32)
        acc = acc + lax.dot_general(xr[...], wr[...], (((1,), (1,)), ((), ())),
                                    preferred_element_type=jnp.float32)
        acc = acc + bl[...]
        if relu:
            acc = jnp.maximum(acc, 0.0)
        o[...] = acc

    d0 = degp[0, :, 0:1]
    d1 = degp[1, :, 0:1]
    return pl.pallas_call(
        body,
        grid=grid,
        in_specs=[
            pl.BlockSpec((BLK, D), lambda i: (i, 0)),
            pl.BlockSpec((BLK, D), lambda i: (i, 0)),
            pl.BlockSpec((BLK, 1), lambda i: (i, 0)),
            pl.BlockSpec((BLK, 1), lambda i: (i, 0)),
            pl.BlockSpec((BLK, D), lambda i: (i, 0)),
            pl.BlockSpec((D, D), lambda i: (0, 0)),
            pl.BlockSpec((1, D), lambda i: (0, 0)),
            pl.BlockSpec((D, D), lambda i: (0, 0)),
        ],
        out_specs=pl.BlockSpec((BLK, D), lambda i: (i, 0)),
        out_shape=jax.ShapeDtypeStruct((N_NODES, D), jnp.float32),
    )(aggp[0], aggp[1], d0, d1, x_in, W_l, b_l.reshape(1, D), W_r)


def kernel(x, edge_index, W1_l, b1_l, W1_r, W2_l, b2_l, W2_r):
    ei = edge_index.astype(jnp.int32)
    packed2 = (ei[0] + ei[1] * 65536).reshape(NW, NCHUNK, CHUNK)
    dst2 = ei[1].reshape(NW, NCHUNK, CHUNK)
    zrow = jnp.zeros((STAGE, D), jnp.float32)
    ones2 = jnp.ones((CHUNK, D), jnp.float32)

    degp = _sc_deg(dst2, zrow, ones2)
    aggp1 = _sc_agg(x, packed2, zrow)
    h = _tc_layer(aggp1, degp, x, W1_l, b1_l, W1_r, relu=True)
    aggp2 = _sc_agg(h, packed2, zrow)
    out = _tc_layer(aggp2, degp, h, W2_l, b2_l, W2_r, relu=False)
    return out


# speedup vs baseline: 8.7139x; 1.0228x over previous
import functools

import jax
import jax.numpy as jnp
from jax import lax
from jax.experimental import pallas as pl
from jax.experimental.pallas import tpu as pltpu
from jax.experimental.pallas import tpu_sc as plsc

N_NODES = 10000
N_EDGES = 320000
D = 128
L = 16

NC = 2
NS = 16
NW = NC * NS
CHUNK = 96
NCHUNK = 105
EPW = NCHUNK * CHUNK
N_DUMP = 16
N_ACC = N_NODES + N_DUMP
INIT_TILES = 10
ROWS_PER_TILE = N_NODES // INIT_TILES
STAGE = 40


def _sc_deg(dst2, zdeg, ones2):
    mesh = plsc.VectorSubcoreMesh(core_axis_name="c", subcore_axis_name="s")

    @functools.partial(
        pl.kernel,
        out_type=jax.ShapeDtypeStruct((NC, N_NODES, D), jnp.float32),
        mesh=mesh,
        scratch_types=[
            pltpu.VMEM((NCHUNK, CHUNK), jnp.int32),
            pltpu.VMEM((CHUNK, D), jnp.float32),
            pltpu.VMEM((STAGE, D), jnp.float32),
            pltpu.VMEM_SHARED((N_ACC, D), jnp.float32),
        ],
    )
    def k(dst_hbm, zdeg_hbm, ones_hbm, deg_out, didx, ones_v, dstage,
          deg_sh):
        cid = lax.axis_index("c")
        sid = lax.axis_index("s")
        wid = cid * NS + sid

        @pl.when(sid < INIT_TILES)
        def _():
            pltpu.sync_copy(zdeg_hbm, dstage)
            for c in range(ROWS_PER_TILE // STAGE):
                pltpu.sync_copy(
                    dstage,
                    deg_sh.at[pl.ds(sid * ROWS_PER_TILE + c * STAGE,
                                    STAGE)])

            @pl.when(sid == 0)
            def _():
                pltpu.sync_copy(dstage.at[pl.ds(0, N_DUMP)],
                                deg_sh.at[pl.ds(N_NODES, N_DUMP)])

        pltpu.sync_copy(ones_hbm, ones_v)
        pltpu.sync_copy(dst_hbm.at[wid], didx)
        plsc.subcore_barrier()

        def step(j, carry):
            pltpu.sync_copy(ones_v, deg_sh.at[didx.at[j]], add=True)
            return carry

        lax.fori_loop(0, NCHUNK, step, 0)
        plsc.subcore_barrier()

        @pl.when(sid < INIT_TILES)
        def _():
            for c in range(ROWS_PER_TILE // STAGE):
                base = sid * ROWS_PER_TILE + c * STAGE
                pltpu.sync_copy(deg_sh.at[pl.ds(base, STAGE)], dstage)
                pltpu.sync_copy(dstage, deg_out.at[cid, pl.ds(base, STAGE)])

    return k(dst2, zdeg, ones2)


def _sc_agg(x, packed2, zrow):
    mesh = plsc.VectorSubcoreMesh(core_axis_name="c", subcore_axis_name="s")

    @functools.partial(
        pl.kernel,
        out_type=jax.ShapeDtypeStruct((NC, N_NODES, D), jnp.float32),
        mesh=mesh,
        scratch_types=[
            pltpu.VMEM((NCHUNK, CHUNK), jnp.int32),
            pltpu.VMEM((2, CHUNK), jnp.int32),
            pltpu.VMEM((2, CHUNK), jnp.int32),
            pltpu.VMEM((CHUNK, D), jnp.float32),
            pltpu.VMEM((CHUNK, D), jnp.float32),
            pltpu.VMEM_SHARED((N_ACC, D), jnp.float32),
            pltpu.SemaphoreType.DMA,
            pltpu.SemaphoreType.DMA,
        ],
    )
    def k(x_hbm, pk_hbm, zrow_hbm, agg_out, pidx, usrc, udst, rows0,
          rows1, agg_sh, gsem0, gsem1):
        cid = lax.axis_index("c")
        sid = lax.axis_index("s")
        wid = cid * NS + sid

        def unpack(j, b):
            for kk in range(CHUNK // L):
                v = pidx[j, pl.ds(kk * L, L)]
                usrc[b, pl.ds(kk * L, L)] = lax.bitwise_and(v, 0xFFFF)
                udst[b, pl.ds(kk * L, L)] = lax.shift_right_logical(v, 16)

        @pl.when(sid < INIT_TILES)
        def _():
            pltpu.sync_copy(zrow_hbm, rows0.at[pl.ds(0, STAGE)])
            for c in range(ROWS_PER_TILE // STAGE):
                pltpu.sync_copy(
                    rows0.at[pl.ds(0, STAGE)],
                    agg_sh.at[pl.ds(sid * ROWS_PER_TILE + c * STAGE, STAGE)])

            @pl.when(sid == 0)
            def _():
                pltpu.sync_copy(rows0.at[pl.ds(0, N_DUMP)],
                                agg_sh.at[pl.ds(N_NODES, N_DUMP)])

        pltpu.sync_copy(pk_hbm.at[wid], pidx)
        unpack(0, 0)
        unpack(1, 1)
        pltpu.async_copy(x_hbm.at[usrc.at[0]], rows0, gsem0)
        pltpu.async_copy(x_hbm.at[usrc.at[1]], rows1, gsem1)
        plsc.subcore_barrier()

        def step(i, carry):
            for b, (rb, sb) in enumerate(((rows0, gsem0), (rows1, gsem1))):
                j = 2 * i + b
                pltpu.make_async_copy(x_hbm.at[usrc.at[b]], rb, sb).wait()
                pltpu.sync_copy(rb, agg_sh.at[udst.at[b]], add=True)
                jn = jnp.minimum(j + 2, NCHUNK - 1)
                unpack(jn, b)
                pltpu.async_copy(x_hbm.at[usrc.at[b]], rb, sb)
            return carry

        lax.fori_loop(0, NCHUNK // 2, step, 0)
        pltpu.make_async_copy(x_hbm.at[usrc.at[0]], rows0, gsem0).wait()
        pltpu.sync_copy(rows0, agg_sh.at[udst.at[0]], add=True)
        pltpu.make_async_copy(x_hbm.at[usrc.at[1]], rows1, gsem1).wait()
        plsc.subcore_barrier()

        @pl.when(sid < INIT_TILES)
        def _():
            for c in range(ROWS_PER_TILE // STAGE):
                base = sid * ROWS_PER_TILE + c * STAGE
                pltpu.sync_copy(agg_sh.at[pl.ds(base, STAGE)],
                                rows0.at[pl.ds(0, STAGE)])
                pltpu.sync_copy(rows0.at[pl.ds(0, STAGE)],
                                agg_out.at[cid, pl.ds(base, STAGE)])

    return k(x, packed2, zrow)


def _tc_layer(aggp, degp, x_in, W_l, b_l, W_r, relu):
    BLK = 1000
    grid = (N_NODES // BLK,)

    def body(a0, a1, d0, d1, xr, wl, bl, wr, o):
        deg = jnp.maximum(d0[...] + d1[...], 1.0)
        agg = (a0[...] + a1[...]) / deg
        acc = lax.dot_general(agg, wl[...], (((1,), (1,)), ((), ())),
                              preferred_element_type=jnp.float32)
        acc = acc + lax.dot_general(xr[...], wr[...], (((1,), (1,)), ((), ())),
                                    preferred_element_type=jnp.float32)
        acc = acc + bl[...]
        if relu:
            acc = jnp.maximum(acc, 0.0)
        o[...] = acc

    d0 = degp[0, :, 0:1]
    d1 = degp[1, :, 0:1]
    return pl.pallas_call(
        body,
        grid=grid,
        in_specs=[
            pl.BlockSpec((BLK, D), lambda i: (i, 0)),
            pl.BlockSpec((BLK, D), lambda i: (i, 0)),
            pl.BlockSpec((BLK, 1), lambda i: (i, 0)),
            pl.BlockSpec((BLK, 1), lambda i: (i, 0)),
            pl.BlockSpec((BLK, D), lambda i: (i, 0)),
            pl.BlockSpec((D, D), lambda i: (0, 0)),
            pl.BlockSpec((1, D), lambda i: (0, 0)),
            pl.BlockSpec((D, D), lambda i: (0, 0)),
        ],
        out_specs=pl.BlockSpec((BLK, D), lambda i: (i, 0)),
        out_shape=jax.ShapeDtypeStruct((N_NODES, D), jnp.float32),
    )(aggp[0], aggp[1], d0, d1, x_in, W_l, b_l.reshape(1, D), W_r)


def kernel(x, edge_index, W1_l, b1_l, W1_r, W2_l, b2_l, W2_r):
    ei = edge_index.astype(jnp.int32)
    pad = NW * EPW - N_EDGES
    pidx = jnp.arange(pad, dtype=jnp.int32)
    src = jnp.concatenate([ei[0], pidx % N_NODES])
    dst = jnp.concatenate([ei[1], N_NODES + pidx % N_DUMP])
    packed2 = (src + dst * 65536).reshape(NW, NCHUNK, CHUNK)
    dst2 = dst.reshape(NW, NCHUNK, CHUNK)
    zrow = jnp.zeros((STAGE, D), jnp.float32)
    ones2 = jnp.ones((CHUNK, D), jnp.float32)

    degp = _sc_deg(dst2, zrow, ones2)
    aggp1 = _sc_agg(x, packed2, zrow)
    h = _tc_layer(aggp1, degp, x, W1_l, b1_l, W1_r, relu=True)
    aggp2 = _sc_agg(h, packed2, zrow)
    out = _tc_layer(aggp2, degp, h, W2_l, b2_l, W2_r, relu=False)
    return out


# speedup vs baseline: 9.0952x; 1.0437x over previous
import functools

import jax
import jax.numpy as jnp
from jax import lax
from jax.experimental import pallas as pl
from jax.experimental.pallas import tpu as pltpu
from jax.experimental.pallas import tpu_sc as plsc

N_NODES = 10000
N_EDGES = 320000
D = 128
L = 16

NC = 2
NS = 16
NW = NC * NS
CHUNK = 128
NCHUNK = 79
EPW = NCHUNK * CHUNK
N_DUMP = 16
N_ACC = N_NODES + N_DUMP
INIT_TILES = 10
ROWS_PER_TILE = N_NODES // INIT_TILES
STAGE = 40


def _sc_deg(dst2, zdeg, ones2):
    mesh = plsc.VectorSubcoreMesh(core_axis_name="c", subcore_axis_name="s")

    @functools.partial(
        pl.kernel,
        out_type=jax.ShapeDtypeStruct((NC, N_NODES, D), jnp.float32),
        mesh=mesh,
        scratch_types=[
            pltpu.VMEM((NCHUNK, CHUNK), jnp.int32),
            pltpu.VMEM((CHUNK, D), jnp.float32),
            pltpu.VMEM((STAGE, D), jnp.float32),
            pltpu.VMEM_SHARED((N_ACC, D), jnp.float32),
        ],
    )
    def k(dst_hbm, zdeg_hbm, ones_hbm, deg_out, didx, ones_v, dstage,
          deg_sh):
        cid = lax.axis_index("c")
        sid = lax.axis_index("s")
        wid = cid * NS + sid

        @pl.when(sid < INIT_TILES)
        def _():
            pltpu.sync_copy(zdeg_hbm, dstage)
            for c in range(ROWS_PER_TILE // STAGE):
                pltpu.sync_copy(
                    dstage,
                    deg_sh.at[pl.ds(sid * ROWS_PER_TILE + c * STAGE,
                                    STAGE)])

            @pl.when(sid == 0)
            def _():
                pltpu.sync_copy(dstage.at[pl.ds(0, N_DUMP)],
                                deg_sh.at[pl.ds(N_NODES, N_DUMP)])

        pltpu.sync_copy(ones_hbm, ones_v)
        pltpu.sync_copy(dst_hbm.at[wid], didx)
        plsc.subcore_barrier()

        def step(j, carry):
            pltpu.sync_copy(ones_v, deg_sh.at[didx.at[j]], add=True)
            return carry

        lax.fori_loop(0, NCHUNK, step, 0)
        plsc.subcore_barrier()

        @pl.when(sid < INIT_TILES)
        def _():
            for c in range(ROWS_PER_TILE // STAGE):
                base = sid * ROWS_PER_TILE + c * STAGE
                pltpu.sync_copy(deg_sh.at[pl.ds(base, STAGE)], dstage)
                pltpu.sync_copy(dstage, deg_out.at[cid, pl.ds(base, STAGE)])

    return k(dst2, zdeg, ones2)


def _sc_agg(x, packed2, zrow):
    mesh = plsc.VectorSubcoreMesh(core_axis_name="c", subcore_axis_name="s")

    @functools.partial(
        pl.kernel,
        out_type=jax.ShapeDtypeStruct((NC, N_NODES, D), jnp.float32),
        mesh=mesh,
        scratch_types=[
            pltpu.VMEM((NCHUNK, CHUNK), jnp.int32),
            pltpu.VMEM((2, CHUNK), jnp.int32),
            pltpu.VMEM((2, CHUNK), jnp.int32),
            pltpu.VMEM((CHUNK, D), jnp.float32),
            pltpu.VMEM((CHUNK, D), jnp.float32),
            pltpu.VMEM_SHARED((N_ACC, D), jnp.float32),
            pltpu.SemaphoreType.DMA,
            pltpu.SemaphoreType.DMA,
        ],
    )
    def k(x_hbm, pk_hbm, zrow_hbm, agg_out, pidx, usrc, udst, rows0,
          rows1, agg_sh, gsem0, gsem1):
        cid = lax.axis_index("c")
        sid = lax.axis_index("s")
        wid = cid * NS + sid

        def unpack(j, b):
            for kk in range(CHUNK // L):
                v = pidx[j, pl.ds(kk * L, L)]
                usrc[b, pl.ds(kk * L, L)] = lax.bitwise_and(v, 0xFFFF)
                udst[b, pl.ds(kk * L, L)] = lax.shift_right_logical(v, 16)

        @pl.when(sid < INIT_TILES)
        def _():
            pltpu.sync_copy(zrow_hbm, rows0.at[pl.ds(0, STAGE)])
            for c in range(ROWS_PER_TILE // STAGE):
                pltpu.sync_copy(
                    rows0.at[pl.ds(0, STAGE)],
                    agg_sh.at[pl.ds(sid * ROWS_PER_TILE + c * STAGE, STAGE)])

            @pl.when(sid == 0)
            def _():
                pltpu.sync_copy(rows0.at[pl.ds(0, N_DUMP)],
                                agg_sh.at[pl.ds(N_NODES, N_DUMP)])

        pltpu.sync_copy(pk_hbm.at[wid], pidx)
        unpack(0, 0)
        unpack(1, 1)
        pltpu.async_copy(x_hbm.at[usrc.at[0]], rows0, gsem0)
        pltpu.async_copy(x_hbm.at[usrc.at[1]], rows1, gsem1)
        plsc.subcore_barrier()

        def step(i, carry):
            for b, (rb, sb) in enumerate(((rows0, gsem0), (rows1, gsem1))):
                j = 2 * i + b
                pltpu.make_async_copy(x_hbm.at[usrc.at[b]], rb, sb).wait()
                pltpu.sync_copy(rb, agg_sh.at[udst.at[b]], add=True)
                jn = jnp.minimum(j + 2, NCHUNK - 1)
                unpack(jn, b)
                pltpu.async_copy(x_hbm.at[usrc.at[b]], rb, sb)
            return carry

        lax.fori_loop(0, NCHUNK // 2, step, 0)
        pltpu.make_async_copy(x_hbm.at[usrc.at[0]], rows0, gsem0).wait()
        pltpu.sync_copy(rows0, agg_sh.at[udst.at[0]], add=True)
        pltpu.make_async_copy(x_hbm.at[usrc.at[1]], rows1, gsem1).wait()
        plsc.subcore_barrier()

        @pl.when(sid < INIT_TILES)
        def _():
            for c in range(ROWS_PER_TILE // STAGE):
                base = sid * ROWS_PER_TILE + c * STAGE
                pltpu.sync_copy(agg_sh.at[pl.ds(base, STAGE)],
                                rows0.at[pl.ds(0, STAGE)])
                pltpu.sync_copy(rows0.at[pl.ds(0, STAGE)],
                                agg_out.at[cid, pl.ds(base, STAGE)])

    return k(x, packed2, zrow)


def _tc_layer(aggp, degp, x_in, W_l, b_l, W_r, relu):
    BLK = 1000
    grid = (N_NODES // BLK,)

    def body(a0, a1, d0, d1, xr, wl, bl, wr, o):
        deg = jnp.maximum(d0[...] + d1[...], 1.0)
        agg = (a0[...] + a1[...]) / deg
        acc = lax.dot_general(agg, wl[...], (((1,), (1,)), ((), ())),
                              preferred_element_type=jnp.float32)
        acc = acc + lax.dot_general(xr[...], wr[...], (((1,), (1,)), ((), ())),
                                    preferred_element_type=jnp.float32)
        acc = acc + bl[...]
        if relu:
            acc = jnp.maximum(acc, 0.0)
        o[...] = acc

    d0 = degp[0, :, 0:1]
    d1 = degp[1, :, 0:1]
    return pl.pallas_call(
        body,
        grid=grid,
        in_specs=[
            pl.BlockSpec((BLK, D), lambda i: (i, 0)),
            pl.BlockSpec((BLK, D), lambda i: (i, 0)),
            pl.BlockSpec((BLK, 1), lambda i: (i, 0)),
            pl.BlockSpec((BLK, 1), lambda i: (i, 0)),
            pl.BlockSpec((BLK, D), lambda i: (i, 0)),
            pl.BlockSpec((D, D), lambda i: (0, 0)),
            pl.BlockSpec((1, D), lambda i: (0, 0)),
            pl.BlockSpec((D, D), lambda i: (0, 0)),
        ],
        out_specs=pl.BlockSpec((BLK, D), lambda i: (i, 0)),
        out_shape=jax.ShapeDtypeStruct((N_NODES, D), jnp.float32),
    )(aggp[0], aggp[1], d0, d1, x_in, W_l, b_l.reshape(1, D), W_r)


def kernel(x, edge_index, W1_l, b1_l, W1_r, W2_l, b2_l, W2_r):
    ei = edge_index.astype(jnp.int32)
    pad = NW * EPW - N_EDGES
    pidx = jnp.arange(pad, dtype=jnp.int32)
    src = jnp.concatenate([ei[0], pidx % N_NODES])
    dst = jnp.concatenate([ei[1], N_NODES + pidx % N_DUMP])
    packed2 = (src + dst * 65536).reshape(NW, NCHUNK, CHUNK)
    dst2 = dst.reshape(NW, NCHUNK, CHUNK)
    zrow = jnp.zeros((STAGE, D), jnp.float32)
    ones2 = jnp.ones((CHUNK, D), jnp.float32)

    degp = _sc_deg(dst2, zrow, ones2)
    aggp1 = _sc_agg(x, packed2, zrow)
    h = _tc_layer(aggp1, degp, x, W1_l, b1_l, W1_r, relu=True)
    aggp2 = _sc_agg(h, packed2, zrow)
    out = _tc_layer(aggp2, degp, h, W2_l, b2_l, W2_r, relu=False)
    return out
